# baseline (speedup 1.0000x reference)
import functools

import jax
import jax.numpy as jnp
from jax import lax
from jax.experimental import pallas as pl
from jax.experimental.pallas import tpu as pltpu
from jax.experimental.pallas import tpu_sc as plsc

_N = 50000
_E = 600000
_D = 128
_NT = 6

_NC = 2
_NS = 16
_NW = _NC * _NS
_NPAD = 50176
_RPT = _NPAD // _NS
_NBB = 74
_EPAD = _NW * _NBB * 256
_NS_PIPE = 4


def _dma_start(src, dst, sem):
    pltpu.make_async_copy(src, dst, sem).start()


def _dma_wait(src, dst, sem):
    pltpu.make_async_copy(src, dst, sem).wait()


@functools.cache
def _get_seg_sum():
    mesh = plsc.VectorSubcoreMesh(core_axis_name="c", subcore_axis_name="s")
    return functools.partial(
        pl.kernel,
        mesh=mesh,
        out_type=[
            jax.ShapeDtypeStruct((_NC * 4 * _NPAD, 32), jnp.bfloat16),
            jax.ShapeDtypeStruct((_NW * _NPAD,), jnp.float32),
        ],
        scratch_types=[
            pltpu.VMEM((16, 256), jnp.int32),
            pltpu.VMEM((16, 256), jnp.int32),
            [pltpu.VMEM((256, 32), jnp.bfloat16) for _ in range(_NS_PIPE)],
            pltpu.VMEM((_NPAD,), jnp.float32),
            pltpu.VMEM_SHARED((_NPAD, 32), jnp.bfloat16),
            [pltpu.SemaphoreType.DMA for _ in range(_NS_PIPE)],
            [pltpu.SemaphoreType.DMA for _ in range(_NS_PIPE)],
        ],
        compiler_params=pltpu.CompilerParams(
            needs_layout_passes=False, use_tc_tiling_on_sc=False),
    )(_seg_sum_body)


def _seg_sum_body(x2, s4, d4, agg_out, cnt_out, srcb, dstb, rows, cntv,
                  shared, semG, semS):
    c = lax.axis_index("c")
    s = lax.axis_index("s")
    wid = c * _NS + s
    myrow0 = s * _RPT

    zeros32 = jnp.zeros((32,), jnp.bfloat16)
    ones32 = jnp.ones((32,), jnp.bfloat16)

    def _fill(ref, val):
        def _f(r, carry):
            ref[r, pl.ds(0, 32)] = val
            return carry
        lax.fori_loop(0, 256, _f, 0)

    NP = _NS_PIPE
    if True:
        def _gather(j, b):
            pltpu.async_copy(x2.at[srcb.at[j]], rows[b], semG[b])

        def _gwait(j, b):
            _dma_wait(x2.at[srcb.at[j]], rows[b], semG[b])

        def _scat(j, b, rb):
            pltpu.async_copy(rows[rb], shared.at[dstb.at[j]], semS[b],
                             add=True)

        def _swait(j, b, rb):
            _dma_wait(rows[rb], shared.at[dstb.at[j]], semS[b])

        for q in range(4):
            _fill(rows[0], zeros32)
            for i in range(12):
                pltpu.sync_copy(rows[0],
                                shared.at[pl.ds(myrow0 + i * 256, 256)])
            pltpu.sync_copy(rows[0].at[pl.ds(0, 64)],
                            shared.at[pl.ds(myrow0 + 12 * 256, 64)])
            if q == 0:
                def _zc(i, carry):
                    cntv[pl.ds(i * 16, 16)] = jnp.zeros((16,), jnp.float32)
                    return carry
                lax.fori_loop(0, _NPAD // 16, _zc, 0)
            plsc.subcore_barrier()

            off = 0
            for nb in (16, 16, 16, 16, 10):
                pltpu.sync_copy(d4.at[wid, pl.ds(off, nb)],
                                dstb.at[pl.ds(0, nb)])
                pltpu.sync_copy(s4.at[wid, pl.ds(off, nb)],
                                srcb.at[pl.ds(0, nb)])

                def _sc4(j, carry):
                    for l in range(16):
                        v = srcb[j, pl.ds(l * 16, 16)]
                        srcb[j, pl.ds(l * 16, 16)] = v * 4 + q
                    return carry
                lax.fori_loop(0, nb, _sc4, 0)

                for b in range(NP):
                    _gather(b, b)

                def _chunk(k, carry):
                    jp = NP * k
                    for b in range(NP):
                        _gwait(jp + b, b)
                        _scat(jp + b, b, b)
                    for b in range(NP):
                        _swait(jp + b, b, b)
                        _gather(jp + NP + b, b)
                    return carry

                nchunk = (nb - 2) // NP - 1
                lax.fori_loop(0, nchunk, _chunk, 0)

                jt = (nchunk + 1) * NP
                for b in range(NP):
                    _gwait(jt - NP + b, b)
                    _scat(jt - NP + b, b, b)
                for b in range(nb - jt):
                    _swait(jt - NP + b, b, b)
                    _gather(jt + b, b)
                    _gwait(jt + b, b)
                    _scat(jt + b, b, b)
                for b in range(nb - jt, NP):
                    _swait(jt - NP + b, b, b)
                for b in range(nb - jt):
                    _swait(jt + b, b, b)

                if q == 0:
                    ones16 = jnp.ones((16,), jnp.float32)

                    def _cnt(j, carry):
                        for l in range(16):
                            idxv = dstb[j, pl.ds(l * 16, 16)]
                            plsc.addupdate_scatter(cntv, [idxv], ones16)
                        return carry
                    lax.fori_loop(0, nb, _cnt, 0)
                off += nb

            if q == 0:
                pltpu.sync_copy(cntv, cnt_out.at[pl.ds(wid * _NPAD, _NPAD)])
            plsc.subcore_barrier()

            dst_off = (c * 4 + q) * _NPAD + myrow0
            pltpu.sync_copy(shared.at[pl.ds(myrow0, _RPT)],
                            agg_out.at[pl.ds(dst_off, _RPT)])
            plsc.subcore_barrier()


def _fuse_body(wl_ref, wr_ref, wp_ref, bl_ref, bp_ref, wlp_ref, wrp_ref, bf_ref):
    wp = wp_ref[0]
    wlp_ref[0] = jnp.dot(wl_ref[0], wp, preferred_element_type=jnp.float32)
    wrp_ref[0] = jnp.dot(wr_ref[0], wp, preferred_element_type=jnp.float32)
    bf_ref[0] = jnp.dot(bl_ref[0], wp, preferred_element_type=jnp.float32) + bp_ref[0]


_fuse = pl.pallas_call(
    _fuse_body,
    grid=(_NT,),
    in_specs=[
        pl.BlockSpec((1, _D, _D), lambda i: (i, 0, 0)),
        pl.BlockSpec((1, _D, _D), lambda i: (i, 0, 0)),
        pl.BlockSpec((1, _D, _D), lambda i: (i, 0, 0)),
        pl.BlockSpec((1, 1, _D), lambda i: (i, 0, 0)),
        pl.BlockSpec((1, 1, _D), lambda i: (i, 0, 0)),
    ],
    out_specs=[
        pl.BlockSpec((1, _D, _D), lambda i: (i, 0, 0)),
        pl.BlockSpec((1, _D, _D), lambda i: (i, 0, 0)),
        pl.BlockSpec((1, 1, _D), lambda i: (i, 0, 0)),
    ],
    out_shape=[
        jax.ShapeDtypeStruct((_NT, _D, _D), jnp.float32),
        jax.ShapeDtypeStruct((_NT, _D, _D), jnp.float32),
        jax.ShapeDtypeStruct((_NT, 1, _D), jnp.float32),
    ],
)

_R = 2000


def _mm_body(ap_ref, cn_ref, x_ref, wlp_ref, wrp_ref, bf_ref, out_ref):
    ap = ap_ref[...].astype(jnp.float32)
    cnt = jnp.sum(cn_ref[...], axis=1)
    inv = (1.0 / jnp.maximum(cnt, 1.0))[:, None]
    p = jnp.dot(x_ref[...], wrp_ref[...], preferred_element_type=jnp.float32)
    p = p + bf_ref[...]
    for q in range(4):
        aq = (ap[0, q] + ap[1, q]) * inv
        p = p + jnp.dot(aq, wlp_ref[32 * q:32 * (q + 1), :],
                        preferred_element_type=jnp.float32)
    m = jnp.max(p, axis=1, keepdims=True)
    e = jnp.exp(p - m)
    out_ref[...] = e / jnp.sum(e, axis=1, keepdims=True)


_mm = pl.pallas_call(
    _mm_body,
    grid=(_N // _R,),
    in_specs=[
        pl.BlockSpec((_NC, 4, _R, 32), lambda i: (0, 0, i, 0)),
        pl.BlockSpec((_R, _NW), lambda i: (i, 0)),
        pl.BlockSpec((_R, _D), lambda i: (i, 0)),
        pl.BlockSpec((_D, _D), lambda i: (0, 0)),
        pl.BlockSpec((_D, _D), lambda i: (0, 0)),
        pl.BlockSpec((1, _D), lambda i: (0, 0)),
    ],
    out_specs=pl.BlockSpec((_R, _D), lambda i: (i, 0)),
    out_shape=jax.ShapeDtypeStruct((_N, _D), jnp.float32),
)


def kernel(x_Path, x_DNS_Host, x_Package_Name, x_IP, x_Hostnames, x_Command,
           x_Port, ei_Path, ei_DNS_Host, ei_IP, ei_Hostnames, ei_Command,
           ei_Port, Wl, Wr, bl, Wp, bp):
    xs = [x_Path, x_DNS_Host, x_IP, x_Hostnames, x_Command, x_Port]
    eis = [ei_Path, ei_DNS_Host, ei_IP, ei_Hostnames, ei_Command, ei_Port]

    Wlp, Wrp, bf = _fuse(Wl, Wr, Wp, bl[:, None, :], bp[:, None, :])

    pad = _EPAD - _E
    outs = []
    for i in range(_NT):
        x2 = xs[i].astype(jnp.bfloat16).reshape(4 * _N, 32)
        src = jnp.concatenate(
            [eis[i][0], jnp.zeros((pad,), jnp.int32)]).reshape(_NW, _NBB, 256)
        dst = jnp.concatenate(
            [eis[i][1], jnp.full((pad,), _N, jnp.int32)]).reshape(_NW, _NBB, 256)
        agg_flat, cnt_flat = _get_seg_sum()(x2, src, dst)
        ap = agg_flat.reshape(_NC, 4, _NPAD, 32)
        cp = cnt_flat.reshape(_NW, _NPAD).T
        out = _mm(ap, cp, xs[i], Wlp[i], Wrp[i], bf[i])
        outs.append(out)
    return tuple(outs)


# speedup vs baseline: 1.2148x; 1.2148x over previous
import functools

import jax
import jax.numpy as jnp
from jax import lax
from jax.experimental import pallas as pl
from jax.experimental.pallas import tpu as pltpu
from jax.experimental.pallas import tpu_sc as plsc

_N = 50000
_E = 600000
_D = 128
_NT = 6

_NC = 2
_NS = 16
_NW = _NC * _NS
_NPAD = 50176
_RPT = _NPAD // _NS
_NBB = 74
_EPAD = _NW * _NBB * 256
_NP = 6


def _dma_wait(src, dst, sem):
    pltpu.make_async_copy(src, dst, sem).wait()


@functools.cache
def _get_seg_sum():
    mesh = plsc.VectorSubcoreMesh(core_axis_name="c", subcore_axis_name="s")
    return functools.partial(
        pl.kernel,
        mesh=mesh,
        out_type=jax.ShapeDtypeStruct((_NC * 5 * _NPAD, 32), jnp.bfloat16),
        scratch_types=[
            pltpu.VMEM((_NBB, 256), jnp.int32),
            pltpu.VMEM((_NBB, 256), jnp.int32),
            [pltpu.VMEM((256, 32), jnp.bfloat16) for _ in range(_NP)],
            pltpu.VMEM_SHARED((_NPAD, 32), jnp.bfloat16),
            [pltpu.SemaphoreType.DMA for _ in range(_NP)],
            [pltpu.SemaphoreType.DMA for _ in range(_NP)],
        ],
        compiler_params=pltpu.CompilerParams(
            needs_layout_passes=False, use_tc_tiling_on_sc=False),
    )(_seg_sum_body)


def _seg_sum_body(x2, s4, d4, agg_out, srcb, dstb, rows, shared, semG, semS):
    c = lax.axis_index("c")
    s = lax.axis_index("s")
    wid = c * _NS + s
    myrow0 = s * _RPT

    zeros32 = jnp.zeros((32,), jnp.bfloat16)
    ones32 = jnp.ones((32,), jnp.bfloat16)

    def _fill(ref, val):
        def _f(r, carry):
            ref[r, pl.ds(0, 32)] = val
            return carry
        lax.fori_loop(0, 256, _f, 0)

    def _gather(j, b):
        pltpu.async_copy(x2.at[srcb.at[j]], rows[b], semG[b])

    def _gwait(j, b):
        _dma_wait(x2.at[srcb.at[j]], rows[b], semG[b])

    def _scat(j, b, rb):
        pltpu.async_copy(rows[rb], shared.at[dstb.at[j]], semS[b], add=True)

    def _swait(j, b, rb):
        _dma_wait(rows[rb], shared.at[dstb.at[j]], semS[b])

    pltpu.sync_copy(d4.at[wid], dstb)
    pltpu.sync_copy(s4.at[wid], srcb)

    def _sc4(j, carry):
        for l in range(16):
            v = srcb[j, pl.ds(l * 16, 16)]
            srcb[j, pl.ds(l * 16, 16)] = v * 4
        return carry
    lax.fori_loop(0, _NBB, _sc4, 0)

    for q in range(5):
        _fill(rows[0], zeros32)
        for i in range(12):
            pltpu.sync_copy(rows[0],
                            shared.at[pl.ds(myrow0 + i * 256, 256)])
        pltpu.sync_copy(rows[0].at[pl.ds(0, 64)],
                        shared.at[pl.ds(myrow0 + 12 * 256, 64)])
        if q == 4:
            _fill(rows[0], ones32)
        plsc.subcore_barrier()

        if q < 4:
            if q > 0:
                def _inc(j, carry):
                    for l in range(16):
                        v = srcb[j, pl.ds(l * 16, 16)]
                        srcb[j, pl.ds(l * 16, 16)] = v + 1
                    return carry
                lax.fori_loop(0, _NBB, _inc, 0)

            for b in range(_NP):
                _gather(b, b)

            def _chunk(k, carry):
                jp = _NP * k
                for b in range(_NP):
                    _gwait(jp + b, b)
                    _scat(jp + b, b, b)
                for b in range(_NP):
                    _swait(jp + b, b, b)
                    _gather(jp + _NP + b, b)
                return carry

            nchunk = (_NBB - 2) // _NP - 1
            lax.fori_loop(0, nchunk, _chunk, 0)

            jt = (nchunk + 1) * _NP
            for b in range(_NP):
                _gwait(jt - _NP + b, b)
                _scat(jt - _NP + b, b, b)
            for b in range(_NBB - jt):
                _swait(jt - _NP + b, b, b)
                _gather(jt + b, b)
                _gwait(jt + b, b)
                _scat(jt + b, b, b)
            for b in range(_NBB - jt, _NP):
                _swait(jt - _NP + b, b, b)
            for b in range(_NBB - jt):
                _swait(jt + b, b, b)
        else:
            def _cbody(j, carry):
                @pl.when(j >= _NP)
                def _():
                    _swait(j - _NP, 0, 0)
                _scat(j, 0, 0)
                return carry

            lax.fori_loop(0, _NBB, _cbody, 0)
            for j in range(_NBB - _NP, _NBB):
                _swait(j, 0, 0)

        plsc.subcore_barrier()

        dst_off = (c * 5 + q) * _NPAD + myrow0
        pltpu.sync_copy(shared.at[pl.ds(myrow0, _RPT)],
                        agg_out.at[pl.ds(dst_off, _RPT)])
        plsc.subcore_barrier()


def _fuse_body(wl_ref, wr_ref, wp_ref, bl_ref, bp_ref, wlp_ref, wrp_ref, bf_ref):
    wp = wp_ref[0]
    wlp_ref[0] = jnp.dot(wl_ref[0], wp, preferred_element_type=jnp.float32)
    wrp_ref[0] = jnp.dot(wr_ref[0], wp, preferred_element_type=jnp.float32)
    bf_ref[0] = jnp.dot(bl_ref[0], wp, preferred_element_type=jnp.float32) + bp_ref[0]


_fuse = pl.pallas_call(
    _fuse_body,
    grid=(_NT,),
    in_specs=[
        pl.BlockSpec((1, _D, _D), lambda i: (i, 0, 0)),
        pl.BlockSpec((1, _D, _D), lambda i: (i, 0, 0)),
        pl.BlockSpec((1, _D, _D), lambda i: (i, 0, 0)),
        pl.BlockSpec((1, 1, _D), lambda i: (i, 0, 0)),
        pl.BlockSpec((1, 1, _D), lambda i: (i, 0, 0)),
    ],
    out_specs=[
        pl.BlockSpec((1, _D, _D), lambda i: (i, 0, 0)),
        pl.BlockSpec((1, _D, _D), lambda i: (i, 0, 0)),
        pl.BlockSpec((1, 1, _D), lambda i: (i, 0, 0)),
    ],
    out_shape=[
        jax.ShapeDtypeStruct((_NT, _D, _D), jnp.float32),
        jax.ShapeDtypeStruct((_NT, _D, _D), jnp.float32),
        jax.ShapeDtypeStruct((_NT, 1, _D), jnp.float32),
    ],
)

_R = 2000


def _mm_body(ap_ref, x_ref, wlp_ref, wrp_ref, bf_ref, out_ref):
    ap = ap_ref[...].astype(jnp.float32)
    cnt = ap[0, 4, :, 0] + ap[1, 4, :, 0]
    inv = (1.0 / jnp.maximum(cnt, 1.0))[:, None]
    p = jnp.dot(x_ref[...], wrp_ref[...], preferred_element_type=jnp.float32)
    p = p + bf_ref[...]
    for q in range(4):
        aq = (ap[0, q] + ap[1, q]) * inv
        p = p + jnp.dot(aq, wlp_ref[32 * q:32 * (q + 1), :],
                        preferred_element_type=jnp.float32)
    m = jnp.max(p, axis=1, keepdims=True)
    e = jnp.exp(p - m)
    out_ref[...] = e / jnp.sum(e, axis=1, keepdims=True)


_mm = pl.pallas_call(
    _mm_body,
    grid=(_N // _R,),
    in_specs=[
        pl.BlockSpec((_NC, 5, _R, 32), lambda i: (0, 0, i, 0)),
        pl.BlockSpec((_R, _D), lambda i: (i, 0)),
        pl.BlockSpec((_D, _D), lambda i: (0, 0)),
        pl.BlockSpec((_D, _D), lambda i: (0, 0)),
        pl.BlockSpec((1, _D), lambda i: (0, 0)),
    ],
    out_specs=pl.BlockSpec((_R, _D), lambda i: (i, 0)),
    out_shape=jax.ShapeDtypeStruct((_N, _D), jnp.float32),
)


def kernel(x_Path, x_DNS_Host, x_Package_Name, x_IP, x_Hostnames, x_Command,
           x_Port, ei_Path, ei_DNS_Host, ei_IP, ei_Hostnames, ei_Command,
           ei_Port, Wl, Wr, bl, Wp, bp):
    xs = [x_Path, x_DNS_Host, x_IP, x_Hostnames, x_Command, x_Port]
    eis = [ei_Path, ei_DNS_Host, ei_IP, ei_Hostnames, ei_Command, ei_Port]

    Wlp, Wrp, bf = _fuse(Wl, Wr, Wp, bl[:, None, :], bp[:, None, :])

    pad = _EPAD - _E
    outs = []
    for i in range(_NT):
        x2 = xs[i].astype(jnp.bfloat16).reshape(4 * _N, 32)
        src = jnp.concatenate(
            [eis[i][0], jnp.zeros((pad,), jnp.int32)]).reshape(_NW, _NBB, 256)
        dst = jnp.concatenate(
            [eis[i][1], jnp.full((pad,), _N, jnp.int32)]).reshape(_NW, _NBB, 256)
        agg_flat = _get_seg_sum()(x2, src, dst)
        ap = agg_flat.reshape(_NC, 5, _NPAD, 32)
        out = _mm(ap, xs[i], Wlp[i], Wrp[i], bf[i])
        outs.append(out)
    return tuple(outs)
